# Initial kernel scaffold; baseline (speedup 1.0000x reference)
import jax
import jax.numpy as jnp
from jax.experimental import pallas as pl


def kernel(features, W_in1, b_in1, W_in2, b_in2, g_nin, beta_nin, codebook, W_out1, b_out1, W_out2, b_out2, g_nout, beta_nout):
    raise NotImplementedError("write your pallas kernel here")


# speedup vs baseline: 1.4070x; 1.4070x over previous
import functools

import jax
import jax.numpy as jnp
from jax import lax
from jax.experimental import pallas as pl
from jax.experimental.pallas import tpu as pltpu
from jax.experimental.pallas import tpu_sc as plsc

_B, _T, _D, _CD, _K = 16, 1024, 768, 256, 8192
_N = _B * _T

_R1 = 256
_G1 = _N // _R1


def _front_body(feat, w1, b1, w2, b2, g, beta, cbt, idx_out, enc_out, e2_scr):
    @pl.when(pl.program_id(0) == 0)
    def _():
        c = cbt[...]
        e2_scr[...] = jnp.sum(c * c, axis=0, keepdims=True)

    x = feat[...]
    h = jnp.maximum(jnp.dot(x, w1[...], preferred_element_type=jnp.float32)
                    + b1[...], 0.0)
    h = jnp.dot(h, w2[...], preferred_element_type=jnp.float32) + b2[...]
    mu = jnp.mean(h, axis=1, keepdims=True)
    var = jnp.mean((h - mu) ** 2, axis=1, keepdims=True)
    flat = (h - mu) / jnp.sqrt(var + 1e-5) * g[...] + beta[...]

    x2 = jnp.sum(flat * flat, axis=1, keepdims=True)
    m = jnp.dot(flat, cbt[...], preferred_element_type=jnp.float32)
    d = (x2 + e2_scr[...]) - 2.0 * m
    dmin = jnp.min(d, axis=1, keepdims=True)
    iota = lax.broadcasted_iota(jnp.int32, (_R1, _K), 1)
    idxs = jnp.min(jnp.where(d == dmin, iota, _K), axis=1)
    idx_out[0, 0, :] = idxs
    enc_out[...] = (iota == idxs[:, None]).astype(jnp.float32)


_front = pl.pallas_call(
    _front_body,
    grid=(_G1,),
    in_specs=[
        pl.BlockSpec((_R1, _D), lambda i: (i, 0)),
        pl.BlockSpec((_D, _D), lambda i: (0, 0)),
        pl.BlockSpec((1, _D), lambda i: (0, 0)),
        pl.BlockSpec((_D, _CD), lambda i: (0, 0)),
        pl.BlockSpec((1, _CD), lambda i: (0, 0)),
        pl.BlockSpec((1, _CD), lambda i: (0, 0)),
        pl.BlockSpec((1, _CD), lambda i: (0, 0)),
        pl.BlockSpec((_CD, _K), lambda i: (0, 0)),
    ],
    out_specs=[
        pl.BlockSpec((1, 1, _R1), lambda i: (i, 0, 0)),
        pl.BlockSpec((_R1, _K), lambda i: (i, 0)),
    ],
    out_shape=[
        jax.ShapeDtypeStruct((_G1, 1, _R1), jnp.int32),
        jax.ShapeDtypeStruct((_N, _K), jnp.float32),
    ],
    scratch_shapes=[pltpu.VMEM((1, _K), jnp.float32)],
    compiler_params=pltpu.CompilerParams(dimension_semantics=("arbitrary",)),
)

_NC, _NS = 2, 16
_NW = _NC * _NS
_RPW = _N // _NW
_CH = 128
_NCH = _RPW // _CH


def _gather_body(cb_hbm, idx_hbm, out_hbm, idx_v, rows_v, sem):
    wid = lax.axis_index("s") * _NC + lax.axis_index("c")
    for ch in range(_NCH):
        base = wid * _RPW + ch * _CH
        pltpu.sync_copy(idx_hbm.at[pl.ds(base, _CH)], idx_v)
        pltpu.async_copy(cb_hbm.at[idx_v], rows_v, sem).wait()
        pltpu.sync_copy(rows_v, out_hbm.at[pl.ds(base, _CH)])


@functools.cache
def _build_gather():
    return functools.partial(
        pl.kernel,
        out_type=jax.ShapeDtypeStruct((_N, _CD), jnp.float32),
        mesh=plsc.VectorSubcoreMesh(core_axis_name="c", subcore_axis_name="s"),
        scratch_types=[
            pltpu.VMEM((_CH,), jnp.int32),
            pltpu.VMEM((_CH, _CD), jnp.float32),
            pltpu.SemaphoreType.DMA,
        ],
    )(_gather_body)


def _gather(cb, idx):
    return _build_gather()(cb, idx)

_R3 = 1024
_G3 = _N // _R3


def _back_body(qr, wo1, bo1, wo2, bo2, g, beta, out):
    h = jnp.maximum(jnp.dot(qr[...], wo1[...], preferred_element_type=jnp.float32)
                    + bo1[...], 0.0)
    h = jnp.dot(h, wo2[...], preferred_element_type=jnp.float32) + bo2[...]
    mu = jnp.mean(h, axis=1, keepdims=True)
    var = jnp.mean((h - mu) ** 2, axis=1, keepdims=True)
    out[...] = (h - mu) / jnp.sqrt(var + 1e-5) * g[...] + beta[...]


_back = pl.pallas_call(
    _back_body,
    grid=(_G3,),
    in_specs=[
        pl.BlockSpec((_R3, _CD), lambda i: (i, 0)),
        pl.BlockSpec((_CD, _D), lambda i: (0, 0)),
        pl.BlockSpec((1, _D), lambda i: (0, 0)),
        pl.BlockSpec((_D, _D), lambda i: (0, 0)),
        pl.BlockSpec((1, _D), lambda i: (0, 0)),
        pl.BlockSpec((1, _D), lambda i: (0, 0)),
        pl.BlockSpec((1, _D), lambda i: (0, 0)),
    ],
    out_specs=pl.BlockSpec((_R3, _D), lambda i: (i, 0)),
    out_shape=jax.ShapeDtypeStruct((_N, _D), jnp.float32),
    compiler_params=pltpu.CompilerParams(dimension_semantics=("arbitrary",)),
)


def kernel(features, W_in1, b_in1, W_in2, b_in2, g_nin, beta_nin, codebook,
           W_out1, b_out1, W_out2, b_out2, g_nout, beta_nout):
    feat = features.reshape(_N, _D)
    cbt = codebook.T
    idx3, enc = _front(feat, W_in1, b_in1.reshape(1, -1), W_in2,
                       b_in2.reshape(1, -1), g_nin.reshape(1, -1),
                       beta_nin.reshape(1, -1), cbt)
    idx_flat = idx3.reshape(_N)
    qr = _gather(codebook, idx_flat)
    q = _back(qr, W_out1, b_out1.reshape(1, -1), W_out2,
              b_out2.reshape(1, -1), g_nout.reshape(1, -1),
              beta_nout.reshape(1, -1))
    return q.reshape(_B, _T, _D), idx_flat.reshape(-1, 1), enc
